# Initial kernel scaffold; baseline (speedup 1.0000x reference)
import jax
import jax.numpy as jnp
from jax.experimental import pallas as pl


def kernel(x, z, x_grid, z_grid, lengthscale_param):
    raise NotImplementedError("write your pallas kernel here")


# speedup vs baseline: 1.4232x; 1.4232x over previous
import functools

import jax
import jax.numpy as jnp
from jax.experimental import pallas as pl
from jax.experimental.pallas import tpu as pltpu

_BM = 512


def _rbf_kernel(sc_ref, xt_ref, z_ref, xg_ref, zg_ref, out_ref):
    s0 = sc_ref[0]
    s1 = sc_ref[1]
    xg = xg_ref[0]
    g0 = xg[:, 0:1] * s0
    g1 = xg[:, 1:2] * s1
    x0 = xt_ref[0, 0:1, :] * s0
    x1 = xt_ref[0, 1:2, :] * s1
    d0 = g0 - x0
    d1 = g1 - x1
    w = jnp.exp(-(d0 * d0 + d1 * d1))
    out_ref[0] = zg_ref[0] + jnp.dot(
        w, z_ref[0], preferred_element_type=jnp.float32)


@jax.jit
def kernel(x, z, x_grid, z_grid, lengthscale_param):
    m, n, dx = x.shape
    dz = z.shape[-1]
    grid_spatial = x_grid.shape[1:-1]
    M = 1
    for s in grid_spatial:
        M *= s

    lengthscale = 1e-5 + jax.nn.softplus(lengthscale_param)
    sc = (1.0 / (lengthscale * jnp.sqrt(2.0))).astype(jnp.float32)

    xt = jnp.swapaxes(x, 1, 2)
    xg_flat = x_grid.reshape(m, M, dx)
    zg_flat = z_grid.reshape(m, M, dz)

    grid = (m, M // _BM)
    out = pl.pallas_call(
        _rbf_kernel,
        grid=grid,
        in_specs=[
            pl.BlockSpec(memory_space=pltpu.SMEM),
            pl.BlockSpec((1, dx, n), lambda b, i: (b, 0, 0)),
            pl.BlockSpec((1, n, dz), lambda b, i: (b, 0, 0)),
            pl.BlockSpec((1, _BM, dx), lambda b, i: (b, i, 0)),
            pl.BlockSpec((1, _BM, dz), lambda b, i: (b, i, 0)),
        ],
        out_specs=pl.BlockSpec((1, _BM, dz), lambda b, i: (b, i, 0)),
        out_shape=jax.ShapeDtypeStruct((m, M, dz), jnp.float32),
    )(sc, xt, z, xg_flat, zg_flat)

    return (x_grid, out.reshape(z_grid.shape))
